# Initial kernel scaffold; baseline (speedup 1.0000x reference)
import jax
import jax.numpy as jnp
from jax.experimental import pallas as pl


def kernel(x, pos, dist, vec_hat, batch, edge_index, e_Win, e_bin, e_Wh, e_bh, e_Wout, e_bout, e_Wrbf, e_Wo, f_Win, f_bin, f_Wh, f_bh, f_Wout, f_bout, f_Wrbf, f_Wo):
    raise NotImplementedError("write your pallas kernel here")


# speedup vs baseline: 5.0033x; 5.0033x over previous
import functools

import jax
import jax.numpy as jnp
from jax import lax
from jax.experimental import pallas as pl
from jax.experimental.pallas import tpu as pltpu
from jax.experimental.pallas import tpu_sc as plsc

_MAXR = 12.0
_NG = 50
_NGRAPH = 64
_NW = 32
_GC = 80
_BLK = 2000

_sc_params = pltpu.CompilerParams(needs_layout_passes=False)


def _sc_mesh():
    return plsc.VectorSubcoreMesh(core_axis_name="c", subcore_axis_name="s")


def _gather_call(xn, src, dst):
    n, embed = xn.shape
    e = src.shape[0]
    epw = e // _NW
    nchunks = epw // _GC

    @functools.partial(
        pl.kernel,
        out_type=(
            jax.ShapeDtypeStruct((e, embed), jnp.float32),
            jax.ShapeDtypeStruct((e, embed), jnp.float32),
        ),
        mesh=_sc_mesh(),
        compiler_params=_sc_params,
        scratch_types=[
            pltpu.VMEM((epw,), jnp.int32),
            pltpu.VMEM((epw,), jnp.int32),
            pltpu.VMEM((_GC, embed), jnp.float32),
            pltpu.VMEM((_GC, embed), jnp.float32),
            pltpu.SemaphoreType.DMA,
            pltpu.SemaphoreType.DMA,
        ],
    )
    def k(xn_hbm, src_hbm, dst_hbm, xsrc_hbm, xdst_hbm,
          sidx_v, didx_v, srows_v, drows_v, ssem, dsem):
        wid = lax.axis_index("s") * 2 + lax.axis_index("c")
        base = wid * epw
        pltpu.sync_copy(src_hbm.at[pl.ds(base, epw)], sidx_v)
        pltpu.sync_copy(dst_hbm.at[pl.ds(base, epw)], didx_v)

        @pl.loop(0, nchunks)
        def _(j):
            off = j * _GC
            cs = pltpu.async_copy(
                xn_hbm.at[sidx_v.at[pl.ds(off, _GC)]], srows_v, ssem)
            cd = pltpu.async_copy(
                xn_hbm.at[didx_v.at[pl.ds(off, _GC)]], drows_v, dsem)
            cs.wait()
            pltpu.sync_copy(srows_v, xsrc_hbm.at[pl.ds(base + off, _GC)])
            cd.wait()
            pltpu.sync_copy(drows_v, xdst_hbm.at[pl.ds(base + off, _GC)])

    return k(xn, src, dst)


def _mlp_body(xs_ref, xd_ref, xe_ref, dist_ref, vh_ref,
              eWin_ref, ebin_ref, eWh_ref, ebh_ref, eWout_ref, ebout_ref,
              eWrbf_ref, eWo_ref,
              fWin_ref, fbin_ref, fWh_ref, fbh_ref, fWout_ref, fbout_ref,
              fWrbf_ref, fWo_ref,
              en_ref, fx_ref, fy_ref, fz_ref):
    embed = xs_ref.shape[1]
    xs = xs_ref[...]
    xd = xd_ref[...]
    xe = xe_ref[...]
    d = dist_ref[...]
    step = _MAXR / (_NG - 1)
    offs = lax.broadcasted_iota(jnp.int32, (1, _NG), 1).astype(jnp.float32) * step
    coeff = -0.5 / (step * step)
    dd = d - offs
    rbf = jnp.exp(coeff * dd * dd)

    def branch(Win_ref, bin_ref, Wh_ref, bh_ref, Wout_ref, bout_ref,
               Wrbf_ref, Wo_ref):
        W = Win_ref[...]
        a = (jnp.dot(xs, W[0:embed], preferred_element_type=jnp.float32)
             + jnp.dot(xd, W[embed:2 * embed], preferred_element_type=jnp.float32)
             + jnp.dot(xe, W[2 * embed:3 * embed], preferred_element_type=jnp.float32)
             + bin_ref[...])
        h = a * jax.nn.sigmoid(a)
        hh = jnp.dot(h, Wh_ref[...], preferred_element_type=jnp.float32) + bh_ref[...]
        h = h + hh * jax.nn.sigmoid(hh)
        g = jnp.dot(h, Wout_ref[...], preferred_element_type=jnp.float32) + bout_ref[...]
        g = g * jnp.dot(rbf, Wrbf_ref[...], preferred_element_type=jnp.float32)
        return jnp.dot(g, Wo_ref[...], preferred_element_type=jnp.float32)

    se = branch(eWin_ref, ebin_ref, eWh_ref, ebh_ref, eWout_ref, ebout_ref,
                eWrbf_ref, eWo_ref)
    sf = branch(fWin_ref, fbin_ref, fWh_ref, fbh_ref, fWout_ref, fbout_ref,
                fWrbf_ref, fWo_ref)
    vh = vh_ref[...]
    en_ref[...] = se
    fx_ref[...] = sf * vh[:, 0:1]
    fy_ref[...] = sf * vh[:, 1:2]
    fz_ref[...] = sf * vh[:, 2:3]


def _mlp_call(xsrc, xdst, x_full, n, dist2, vh, ws):
    e, embed = xsrc.shape
    grid = e // _BLK
    ntok_blocks = n // _BLK

    def edge_spec(w):
        return pl.BlockSpec((_BLK, w), lambda i: (i, 0))

    def weight_spec(shape):
        if len(shape) == 1:
            return pl.BlockSpec(shape, lambda i: (0,))
        return pl.BlockSpec(shape, lambda i: (0, 0))

    xe_spec = pl.BlockSpec((_BLK, embed), lambda i: (ntok_blocks + i, 0))
    in_specs = [edge_spec(embed), edge_spec(embed), xe_spec,
                edge_spec(1), edge_spec(3)]
    in_specs += [weight_spec(w.shape) for w in ws]
    out_specs = [edge_spec(1)] * 4
    out_shape = [jax.ShapeDtypeStruct((e, 1), jnp.float32)] * 4

    return pl.pallas_call(
        _mlp_body,
        grid=(grid,),
        in_specs=in_specs,
        out_specs=out_specs,
        out_shape=out_shape,
    )(xsrc, xdst, x_full, dist2, vh, *ws)


def _scatter_call(en, fx, fy, fz, src, batch, zeros_f):
    e = src.shape[0]
    n = batch.shape[0]
    epw = e // _NW
    niter = epw // 16

    @functools.partial(
        pl.kernel,
        out_type=(
            jax.ShapeDtypeStruct((_NW, n * 4), jnp.float32),
            jax.ShapeDtypeStruct((_NW, _NGRAPH), jnp.float32),
        ),
        mesh=_sc_mesh(),
        compiler_params=_sc_params,
        scratch_types=[
            pltpu.VMEM((epw,), jnp.int32),
            pltpu.VMEM((n,), jnp.int32),
            pltpu.VMEM((epw,), jnp.float32),
            pltpu.VMEM((epw,), jnp.float32),
            pltpu.VMEM((epw,), jnp.float32),
            pltpu.VMEM((epw,), jnp.float32),
            pltpu.VMEM((n * 4,), jnp.float32),
            pltpu.VMEM((_NGRAPH,), jnp.float32),
        ],
    )
    def k(en_hbm, fx_hbm, fy_hbm, fz_hbm, src_hbm, batch_hbm, zeros_hbm,
          fpart_hbm, epart_hbm,
          src_v, batch_v, ev_v, fx_v, fy_v, fz_v, facc_v, eacc_v):
        wid = lax.axis_index("s") * 2 + lax.axis_index("c")
        base = wid * epw
        pltpu.sync_copy(src_hbm.at[pl.ds(base, epw)], src_v)
        pltpu.sync_copy(batch_hbm, batch_v)
        pltpu.sync_copy(en_hbm.at[pl.ds(base, epw)], ev_v)
        pltpu.sync_copy(fx_hbm.at[pl.ds(base, epw)], fx_v)
        pltpu.sync_copy(fy_hbm.at[pl.ds(base, epw)], fy_v)
        pltpu.sync_copy(fz_hbm.at[pl.ds(base, epw)], fz_v)
        pltpu.sync_copy(zeros_hbm, facc_v)

        zero16 = jnp.zeros((16,), jnp.float32)
        for i in range(_NGRAPH // 16):
            eacc_v[pl.ds(i * 16, 16)] = zero16

        @pl.loop(0, niter)
        def _(j):
            o = j * 16
            idx = src_v[pl.ds(o, 16)]
            seg = plsc.load_gather(batch_v, [idx])
            plsc.addupdate_scatter(eacc_v, [seg], ev_v[pl.ds(o, 16)])
            fi = idx * 4
            plsc.addupdate_scatter(facc_v, [fi], fx_v[pl.ds(o, 16)])
            plsc.addupdate_scatter(facc_v, [fi + 1], fy_v[pl.ds(o, 16)])
            plsc.addupdate_scatter(facc_v, [fi + 2], fz_v[pl.ds(o, 16)])

        pltpu.sync_copy(facc_v, fpart_hbm.at[wid])
        pltpu.sync_copy(eacc_v, epart_hbm.at[wid])

    return k(en, fx, fy, fz, src, batch, zeros_f)


def _reduce_body(fpart_ref, epart_ref, fsum_ref, esum_ref):
    fsum_ref[...] = jnp.sum(fpart_ref[...], axis=0, keepdims=True)
    esum_ref[...] = jnp.sum(epart_ref[...], axis=0, keepdims=True)


def _reduce_call(fpart, epart):
    n4 = fpart.shape[1]
    return pl.pallas_call(
        _reduce_body,
        out_shape=(
            jax.ShapeDtypeStruct((1, n4), jnp.float32),
            jax.ShapeDtypeStruct((1, _NGRAPH), jnp.float32),
        ),
    )(fpart, epart)


def kernel(x, pos, dist, vec_hat, batch, edge_index,
           e_Win, e_bin, e_Wh, e_bh, e_Wout, e_bout, e_Wrbf, e_Wo,
           f_Win, f_bin, f_Wh, f_bh, f_Wout, f_bout, f_Wrbf, f_Wo):
    n = pos.shape[0]
    m = edge_index.shape[1]
    embed = x.shape[1]

    src = edge_index[0]
    dst = edge_index[1]

    xsrc, xdst = _gather_call(x, src, dst)

    ws = (e_Win, e_bin, e_Wh, e_bh, e_Wout, e_bout, e_Wrbf, e_Wo,
          f_Win, f_bin, f_Wh, f_bh, f_Wout, f_bout, f_Wrbf, f_Wo)
    en, fx, fy, fz = _mlp_call(xsrc, xdst, x, n, dist.reshape(m, 1),
                               vec_hat, ws)

    zeros_f = jnp.zeros((n * 4,), jnp.float32)
    fpart, epart = _scatter_call(en.reshape(m), fx.reshape(m), fy.reshape(m),
                                 fz.reshape(m), src, batch, zeros_f)

    fsum, esum = _reduce_call(fpart, epart)
    forces = fsum.reshape(n, 4)[:, :3]
    energy = esum.reshape(_NGRAPH, 1)
    return (energy, forces)


# speedup vs baseline: 5.1250x; 1.0243x over previous
import functools

import jax
import jax.numpy as jnp
from jax import lax
from jax.experimental import pallas as pl
from jax.experimental.pallas import tpu as pltpu
from jax.experimental.pallas import tpu_sc as plsc

_MAXR = 12.0
_NG = 50
_NGRAPH = 64
_NW = 32
_GC = 80
_BLK = 2000

_sc_params = pltpu.CompilerParams(needs_layout_passes=False)


def _sig(v):
    return 0.5 + 0.5 * jnp.tanh(0.5 * v)


def _sc_mesh():
    return plsc.VectorSubcoreMesh(core_axis_name="c", subcore_axis_name="s")


def _gather_call(xn, src, dst):
    n, embed = xn.shape
    dt = xn.dtype
    e = src.shape[0]
    epw = e // _NW
    nchunks = epw // _GC

    @functools.partial(
        pl.kernel,
        out_type=(
            jax.ShapeDtypeStruct((e, embed), dt),
            jax.ShapeDtypeStruct((e, embed), dt),
        ),
        mesh=_sc_mesh(),
        compiler_params=_sc_params,
        scratch_types=[
            pltpu.VMEM((epw,), jnp.int32),
            pltpu.VMEM((epw,), jnp.int32),
            pltpu.VMEM((_GC, embed), dt),
            pltpu.VMEM((_GC, embed), dt),
            pltpu.SemaphoreType.DMA,
            pltpu.SemaphoreType.DMA,
        ],
    )
    def k(xn_hbm, src_hbm, dst_hbm, xsrc_hbm, xdst_hbm,
          sidx_v, didx_v, srows_v, drows_v, ssem, dsem):
        wid = lax.axis_index("s") * 2 + lax.axis_index("c")
        base = wid * epw
        pltpu.sync_copy(src_hbm.at[pl.ds(base, epw)], sidx_v)
        pltpu.sync_copy(dst_hbm.at[pl.ds(base, epw)], didx_v)

        @pl.loop(0, nchunks)
        def _(j):
            off = j * _GC
            cs = pltpu.async_copy(
                xn_hbm.at[sidx_v.at[pl.ds(off, _GC)]], srows_v, ssem)
            cd = pltpu.async_copy(
                xn_hbm.at[didx_v.at[pl.ds(off, _GC)]], drows_v, dsem)
            cs.wait()
            pltpu.sync_copy(srows_v, xsrc_hbm.at[pl.ds(base + off, _GC)])
            cd.wait()
            pltpu.sync_copy(drows_v, xdst_hbm.at[pl.ds(base + off, _GC)])

    return k(xn, src, dst)


def _mlp_body(xs_ref, xd_ref, xe_ref, dist_ref, vh_ref,
              eWin_ref, ebin_ref, eWh_ref, ebh_ref, eWout_ref, ebout_ref,
              eWrbf_ref, eWo_ref,
              fWin_ref, fbin_ref, fWh_ref, fbh_ref, fWout_ref, fbout_ref,
              fWrbf_ref, fWo_ref,
              en_ref, fx_ref, fy_ref, fz_ref):
    embed = xs_ref.shape[1]
    xs = xs_ref[...]
    xd = xd_ref[...]
    xe = xe_ref[...]
    d = dist_ref[...]
    step = _MAXR / (_NG - 1)
    offs = lax.broadcasted_iota(jnp.int32, (1, _NG), 1).astype(jnp.float32) * step
    coeff = -0.5 / (step * step)
    dd = d - offs
    rbf = jnp.exp(coeff * dd * dd)

    def branch(Win_ref, bin_ref, Wh_ref, bh_ref, Wout_ref, bout_ref,
               Wrbf_ref, Wo_ref):
        W = Win_ref[...]
        a = (jnp.dot(xs, W[0:embed], preferred_element_type=jnp.float32)
             + jnp.dot(xd, W[embed:2 * embed], preferred_element_type=jnp.float32)
             + jnp.dot(xe, W[2 * embed:3 * embed], preferred_element_type=jnp.float32)
             + bin_ref[...])
        h = a * _sig(a)
        hh = jnp.dot(h, Wh_ref[...], preferred_element_type=jnp.float32) + bh_ref[...]
        h = h + hh * _sig(hh)
        g = jnp.dot(h, Wout_ref[...], preferred_element_type=jnp.float32) + bout_ref[...]
        g = g * jnp.dot(rbf, Wrbf_ref[...], preferred_element_type=jnp.float32)
        return jnp.dot(g, Wo_ref[...], preferred_element_type=jnp.float32)

    se = branch(eWin_ref, ebin_ref, eWh_ref, ebh_ref, eWout_ref, ebout_ref,
                eWrbf_ref, eWo_ref)
    sf = branch(fWin_ref, fbin_ref, fWh_ref, fbh_ref, fWout_ref, fbout_ref,
                fWrbf_ref, fWo_ref)
    vh = vh_ref[...]
    en_ref[...] = se
    fx_ref[...] = sf * vh[:, 0:1]
    fy_ref[...] = sf * vh[:, 1:2]
    fz_ref[...] = sf * vh[:, 2:3]


def _mlp_call(xsrc, xdst, x_full, n, dist2, vh, ws):
    e, embed = xsrc.shape
    grid = e // _BLK
    ntok_blocks = n // _BLK

    def edge_spec(w):
        return pl.BlockSpec((_BLK, w), lambda i: (i, 0))

    def weight_spec(shape):
        if len(shape) == 1:
            return pl.BlockSpec(shape, lambda i: (0,))
        return pl.BlockSpec(shape, lambda i: (0, 0))

    xe_spec = pl.BlockSpec((_BLK, embed), lambda i: (ntok_blocks + i, 0))
    in_specs = [edge_spec(embed), edge_spec(embed), xe_spec,
                edge_spec(1), edge_spec(3)]
    in_specs += [weight_spec(w.shape) for w in ws]
    out_specs = [edge_spec(1)] * 4
    out_shape = [jax.ShapeDtypeStruct((e, 1), jnp.float32)] * 4

    return pl.pallas_call(
        _mlp_body,
        grid=(grid,),
        in_specs=in_specs,
        out_specs=out_specs,
        out_shape=out_shape,
    )(xsrc, xdst, x_full, dist2, vh, *ws)


def _scatter_call(en, fx, fy, fz, src, batch, zeros_f):
    e = src.shape[0]
    n = batch.shape[0]
    epw = e // _NW
    niter = epw // 16

    @functools.partial(
        pl.kernel,
        out_type=(
            jax.ShapeDtypeStruct((_NW, n * 4), jnp.float32),
            jax.ShapeDtypeStruct((_NW, _NGRAPH), jnp.float32),
        ),
        mesh=_sc_mesh(),
        compiler_params=_sc_params,
        scratch_types=[
            pltpu.VMEM((epw,), jnp.int32),
            pltpu.VMEM((n,), jnp.int32),
            pltpu.VMEM((epw,), jnp.float32),
            pltpu.VMEM((epw,), jnp.float32),
            pltpu.VMEM((epw,), jnp.float32),
            pltpu.VMEM((epw,), jnp.float32),
            pltpu.VMEM((n * 4,), jnp.float32),
            pltpu.VMEM((_NGRAPH,), jnp.float32),
        ],
    )
    def k(en_hbm, fx_hbm, fy_hbm, fz_hbm, src_hbm, batch_hbm, zeros_hbm,
          fpart_hbm, epart_hbm,
          src_v, batch_v, ev_v, fx_v, fy_v, fz_v, facc_v, eacc_v):
        wid = lax.axis_index("s") * 2 + lax.axis_index("c")
        base = wid * epw
        pltpu.sync_copy(src_hbm.at[pl.ds(base, epw)], src_v)
        pltpu.sync_copy(batch_hbm, batch_v)
        pltpu.sync_copy(en_hbm.at[pl.ds(base, epw)], ev_v)
        pltpu.sync_copy(fx_hbm.at[pl.ds(base, epw)], fx_v)
        pltpu.sync_copy(fy_hbm.at[pl.ds(base, epw)], fy_v)
        pltpu.sync_copy(fz_hbm.at[pl.ds(base, epw)], fz_v)
        pltpu.sync_copy(zeros_hbm, facc_v)

        zero16 = jnp.zeros((16,), jnp.float32)
        for i in range(_NGRAPH // 16):
            eacc_v[pl.ds(i * 16, 16)] = zero16

        @pl.loop(0, niter)
        def _(j):
            o = j * 16
            idx = src_v[pl.ds(o, 16)]
            seg = plsc.load_gather(batch_v, [idx])
            plsc.addupdate_scatter(eacc_v, [seg], ev_v[pl.ds(o, 16)])
            fi = idx * 4
            plsc.addupdate_scatter(facc_v, [fi], fx_v[pl.ds(o, 16)])
            plsc.addupdate_scatter(facc_v, [fi + 1], fy_v[pl.ds(o, 16)])
            plsc.addupdate_scatter(facc_v, [fi + 2], fz_v[pl.ds(o, 16)])

        pltpu.sync_copy(facc_v, fpart_hbm.at[wid])
        pltpu.sync_copy(eacc_v, epart_hbm.at[wid])

    return k(en, fx, fy, fz, src, batch, zeros_f)


def _reduce_body(fpart_ref, epart_ref, fsum_ref, esum_ref):
    fsum_ref[...] = jnp.sum(fpart_ref[...], axis=0, keepdims=True)
    esum_ref[...] = jnp.sum(epart_ref[...], axis=0, keepdims=True)


def _reduce_call(fpart, epart):
    n4 = fpart.shape[1]
    return pl.pallas_call(
        _reduce_body,
        out_shape=(
            jax.ShapeDtypeStruct((1, n4), jnp.float32),
            jax.ShapeDtypeStruct((1, _NGRAPH), jnp.float32),
        ),
    )(fpart, epart)


def kernel(x, pos, dist, vec_hat, batch, edge_index,
           e_Win, e_bin, e_Wh, e_bh, e_Wout, e_bout, e_Wrbf, e_Wo,
           f_Win, f_bin, f_Wh, f_bh, f_Wout, f_bout, f_Wrbf, f_Wo):
    n = pos.shape[0]
    m = edge_index.shape[1]
    embed = x.shape[1]

    src = edge_index[0]
    dst = edge_index[1]

    xsrc, xdst = _gather_call(x, src, dst)

    ws = (e_Win, e_bin, e_Wh, e_bh, e_Wout, e_bout, e_Wrbf, e_Wo,
          f_Win, f_bin, f_Wh, f_bh, f_Wout, f_bout, f_Wrbf, f_Wo)
    en, fx, fy, fz = _mlp_call(xsrc, xdst, x, n, dist.reshape(m, 1),
                               vec_hat, ws)

    zeros_f = jnp.zeros((n * 4,), jnp.float32)
    fpart, epart = _scatter_call(en.reshape(m), fx.reshape(m), fy.reshape(m),
                                 fz.reshape(m), src, batch, zeros_f)

    fsum, esum = _reduce_call(fpart, epart)
    forces = fsum.reshape(n, 4)[:, :3]
    energy = esum.reshape(_NGRAPH, 1)
    return (energy, forces)


# speedup vs baseline: 5.4656x; 1.0665x over previous
import functools

import jax
import jax.numpy as jnp
from jax import lax
from jax.experimental import pallas as pl
from jax.experimental.pallas import tpu as pltpu
from jax.experimental.pallas import tpu_sc as plsc

_MAXR = 12.0
_NG = 50
_NGRAPH = 64
_NW = 32
_GC = 80
_BLK = 2000

_sc_params = pltpu.CompilerParams(needs_layout_passes=False)


def _sig(v):
    return 0.5 + 0.5 * jnp.tanh(0.5 * v)


def _sc_mesh():
    return plsc.VectorSubcoreMesh(core_axis_name="c", subcore_axis_name="s")


def _gather_call(xn, src, dst):
    n, embed = xn.shape
    dt = xn.dtype
    e = src.shape[0]
    epw = e // _NW
    nchunks = epw // _GC

    nbuf = 4
    nrounds = (nchunks + nbuf - 1) // nbuf

    @functools.partial(
        pl.kernel,
        out_type=(
            jax.ShapeDtypeStruct((e, embed), dt),
            jax.ShapeDtypeStruct((e, embed), dt),
        ),
        mesh=_sc_mesh(),
        compiler_params=_sc_params,
        scratch_types=(
            [pltpu.VMEM((epw,), jnp.int32)] * 2
            + [pltpu.VMEM((_GC, embed), dt)] * (2 * nbuf)
            + [pltpu.SemaphoreType.DMA] * (4 * nbuf)
        ),
    )
    def k(xn_hbm, src_hbm, dst_hbm, xsrc_hbm, xdst_hbm, *scr):
        sidx_v, didx_v = scr[0], scr[1]
        sbufs = scr[2:2 + nbuf]
        dbufs = scr[2 + nbuf:2 + 2 * nbuf]
        gs = scr[2 + 2 * nbuf:2 + 3 * nbuf]
        gd = scr[2 + 3 * nbuf:2 + 4 * nbuf]
        ws = scr[2 + 4 * nbuf:2 + 5 * nbuf]
        wd = scr[2 + 5 * nbuf:2 + 6 * nbuf]
        wid = lax.axis_index("s") * 2 + lax.axis_index("c")
        base = wid * epw
        pltpu.sync_copy(src_hbm.at[pl.ds(base, epw)], sidx_v)
        pltpu.sync_copy(dst_hbm.at[pl.ds(base, epw)], didx_v)

        def g_start(jj, b):
            off = jj * _GC
            pltpu.async_copy(
                xn_hbm.at[sidx_v.at[pl.ds(off, _GC)]], sbufs[b], gs[b])
            pltpu.async_copy(
                xn_hbm.at[didx_v.at[pl.ds(off, _GC)]], dbufs[b], gd[b])

        def g_wait(b):
            pltpu.make_async_copy(
                xn_hbm.at[pl.ds(0, _GC)], sbufs[b], gs[b]).wait()
            pltpu.make_async_copy(
                xn_hbm.at[pl.ds(0, _GC)], dbufs[b], gd[b]).wait()

        def w_start(jj, b):
            off = jj * _GC
            pltpu.async_copy(
                sbufs[b], xsrc_hbm.at[pl.ds(base + off, _GC)], ws[b])
            pltpu.async_copy(
                dbufs[b], xdst_hbm.at[pl.ds(base + off, _GC)], wd[b])

        def w_wait(b):
            pltpu.make_async_copy(
                sbufs[b], xsrc_hbm.at[pl.ds(base, _GC)], ws[b]).wait()
            pltpu.make_async_copy(
                dbufs[b], xdst_hbm.at[pl.ds(base, _GC)], wd[b]).wait()

        g_start(0, 0)
        g_start(1, 1)

        @pl.loop(0, nrounds)
        def _(r):
            for b in range(nbuf):
                jj = r * nbuf + b

                @pl.when(jj < nchunks)
                def _():
                    b2 = (b + 2) % nbuf

                    @pl.when(jj >= 2)
                    def _():
                        w_wait(b2)

                    @pl.when(jj + 2 < nchunks)
                    def _():
                        g_start(jj + 2, b2)

                    g_wait(b)
                    w_start(jj, b)

        w_wait((nchunks - 2) % nbuf)
        w_wait((nchunks - 1) % nbuf)

    return k(xn, src, dst)


def _mlp_body(xs_ref, xd_ref, xe_ref, dist_ref, vh_ref,
              eWin_ref, ebin_ref, eWh_ref, ebh_ref, eWout_ref, ebout_ref,
              eWrbf_ref, eWo_ref,
              fWin_ref, fbin_ref, fWh_ref, fbh_ref, fWout_ref, fbout_ref,
              fWrbf_ref, fWo_ref,
              en_ref, fx_ref, fy_ref, fz_ref):
    embed = xs_ref.shape[1]
    xs = xs_ref[...]
    xd = xd_ref[...]
    xe = xe_ref[...]
    d = dist_ref[...]
    step = _MAXR / (_NG - 1)
    offs = lax.broadcasted_iota(jnp.int32, (1, _NG), 1).astype(jnp.float32) * step
    coeff = -0.5 / (step * step)
    dd = d - offs
    rbf = jnp.exp(coeff * dd * dd)

    def branch(Win_ref, bin_ref, Wh_ref, bh_ref, Wout_ref, bout_ref,
               Wrbf_ref, Wo_ref):
        W = Win_ref[...]
        a = (jnp.dot(xs, W[0:embed], preferred_element_type=jnp.float32)
             + jnp.dot(xd, W[embed:2 * embed], preferred_element_type=jnp.float32)
             + jnp.dot(xe, W[2 * embed:3 * embed], preferred_element_type=jnp.float32)
             + bin_ref[...])
        h = a * _sig(a)
        hh = jnp.dot(h, Wh_ref[...], preferred_element_type=jnp.float32) + bh_ref[...]
        h = h + hh * _sig(hh)
        g = jnp.dot(h, Wout_ref[...], preferred_element_type=jnp.float32) + bout_ref[...]
        g = g * jnp.dot(rbf, Wrbf_ref[...], preferred_element_type=jnp.float32)
        return jnp.dot(g, Wo_ref[...], preferred_element_type=jnp.float32)

    se = branch(eWin_ref, ebin_ref, eWh_ref, ebh_ref, eWout_ref, ebout_ref,
                eWrbf_ref, eWo_ref)
    sf = branch(fWin_ref, fbin_ref, fWh_ref, fbh_ref, fWout_ref, fbout_ref,
                fWrbf_ref, fWo_ref)
    vh = vh_ref[...]
    b = se.shape[0]
    en_ref[...] = jnp.transpose(se, (1, 0)).reshape(1, 1, b)
    fx_ref[...] = jnp.transpose(sf * vh[:, 0:1], (1, 0)).reshape(1, 1, b)
    fy_ref[...] = jnp.transpose(sf * vh[:, 1:2], (1, 0)).reshape(1, 1, b)
    fz_ref[...] = jnp.transpose(sf * vh[:, 2:3], (1, 0)).reshape(1, 1, b)


def _mlp_call(xsrc, xdst, x_full, n, dist2, vh, ws):
    e, embed = xsrc.shape
    grid = e // _BLK
    ntok_blocks = n // _BLK

    def edge_spec(w):
        return pl.BlockSpec((_BLK, w), lambda i: (i, 0))

    def weight_spec(shape):
        if len(shape) == 1:
            return pl.BlockSpec(shape, lambda i: (0,))
        return pl.BlockSpec(shape, lambda i: (0, 0))

    xe_spec = pl.BlockSpec((_BLK, embed), lambda i: (ntok_blocks + i, 0))
    in_specs = [edge_spec(embed), edge_spec(embed), xe_spec,
                edge_spec(1), edge_spec(3)]
    in_specs += [weight_spec(w.shape) for w in ws]
    out_specs = [pl.BlockSpec((1, 1, _BLK), lambda i: (i, 0, 0))] * 4
    out_shape = [jax.ShapeDtypeStruct((grid, 1, _BLK), jnp.float32)] * 4

    return pl.pallas_call(
        _mlp_body,
        grid=(grid,),
        in_specs=in_specs,
        out_specs=out_specs,
        out_shape=out_shape,
    )(xsrc, xdst, x_full, dist2, vh, *ws)


def _scatter_call(en, fx, fy, fz, src, batch, zeros_f):
    e = src.shape[0]
    n = batch.shape[0]
    epw = e // _NW
    niter = epw // 16

    @functools.partial(
        pl.kernel,
        out_type=(
            jax.ShapeDtypeStruct((_NW, n * 4), jnp.float32),
            jax.ShapeDtypeStruct((_NW, _NGRAPH), jnp.float32),
        ),
        mesh=_sc_mesh(),
        compiler_params=_sc_params,
        scratch_types=[
            pltpu.VMEM((epw,), jnp.int32),
            pltpu.VMEM((n,), jnp.int32),
            pltpu.VMEM((epw,), jnp.float32),
            pltpu.VMEM((epw,), jnp.float32),
            pltpu.VMEM((epw,), jnp.float32),
            pltpu.VMEM((epw,), jnp.float32),
            pltpu.VMEM((n * 4,), jnp.float32),
            pltpu.VMEM((_NGRAPH,), jnp.float32),
        ],
    )
    def k(en_hbm, fx_hbm, fy_hbm, fz_hbm, src_hbm, batch_hbm, zeros_hbm,
          fpart_hbm, epart_hbm,
          src_v, batch_v, ev_v, fx_v, fy_v, fz_v, facc_v, eacc_v):
        wid = lax.axis_index("s") * 2 + lax.axis_index("c")
        base = wid * epw
        pltpu.sync_copy(src_hbm.at[pl.ds(base, epw)], src_v)
        pltpu.sync_copy(batch_hbm, batch_v)
        pltpu.sync_copy(en_hbm.at[pl.ds(base, epw)], ev_v)
        pltpu.sync_copy(fx_hbm.at[pl.ds(base, epw)], fx_v)
        pltpu.sync_copy(fy_hbm.at[pl.ds(base, epw)], fy_v)
        pltpu.sync_copy(fz_hbm.at[pl.ds(base, epw)], fz_v)
        pltpu.sync_copy(zeros_hbm, facc_v)

        zero16 = jnp.zeros((16,), jnp.float32)
        for i in range(_NGRAPH // 16):
            eacc_v[pl.ds(i * 16, 16)] = zero16

        @pl.loop(0, niter)
        def _(j):
            o = j * 16
            idx = src_v[pl.ds(o, 16)]
            seg = plsc.load_gather(batch_v, [idx])
            plsc.addupdate_scatter(eacc_v, [seg], ev_v[pl.ds(o, 16)])
            fi = idx * 4
            plsc.addupdate_scatter(facc_v, [fi], fx_v[pl.ds(o, 16)])
            plsc.addupdate_scatter(facc_v, [fi + 1], fy_v[pl.ds(o, 16)])
            plsc.addupdate_scatter(facc_v, [fi + 2], fz_v[pl.ds(o, 16)])

        pltpu.sync_copy(facc_v, fpart_hbm.at[wid])
        pltpu.sync_copy(eacc_v, epart_hbm.at[wid])

    return k(en, fx, fy, fz, src, batch, zeros_f)


def _reduce_body(fpart_ref, epart_ref, fsum_ref, esum_ref):
    fsum_ref[...] = jnp.sum(fpart_ref[...], axis=0, keepdims=True)
    esum_ref[...] = jnp.sum(epart_ref[...], axis=0, keepdims=True)


def _reduce_call(fpart, epart):
    n4 = fpart.shape[1]
    return pl.pallas_call(
        _reduce_body,
        out_shape=(
            jax.ShapeDtypeStruct((1, n4), jnp.float32),
            jax.ShapeDtypeStruct((1, _NGRAPH), jnp.float32),
        ),
    )(fpart, epart)


def kernel(x, pos, dist, vec_hat, batch, edge_index,
           e_Win, e_bin, e_Wh, e_bh, e_Wout, e_bout, e_Wrbf, e_Wo,
           f_Win, f_bin, f_Wh, f_bh, f_Wout, f_bout, f_Wrbf, f_Wo):
    n = pos.shape[0]
    m = edge_index.shape[1]
    embed = x.shape[1]

    src = edge_index[0]
    dst = edge_index[1]

    xn = lax.slice(x, (0, 0), (n, embed))
    xsrc, xdst = _gather_call(xn, src, dst)

    ws = (e_Win, e_bin, e_Wh, e_bh, e_Wout, e_bout, e_Wrbf, e_Wo,
          f_Win, f_bin, f_Wh, f_bh, f_Wout, f_bout, f_Wrbf, f_Wo)
    en, fx, fy, fz = _mlp_call(xsrc, xdst, x, n, dist.reshape(m, 1),
                               vec_hat, ws)

    zeros_f = jnp.zeros((n * 4,), jnp.float32)
    fpart, epart = _scatter_call(en.reshape(m), fx.reshape(m), fy.reshape(m),
                                 fz.reshape(m), src, batch, zeros_f)

    fsum, esum = _reduce_call(fpart, epart)
    forces = fsum.reshape(n, 4)[:, :3]
    energy = esum.reshape(_NGRAPH, 1)
    return (energy, forces)
